# baseline (speedup 1.0000x reference)
import functools
import math

import jax
import jax.numpy as jnp
from jax import lax
from jax.experimental import pallas as pl
from jax.experimental.pallas import tpu as pltpu
from jax.experimental.pallas import tpu_sc as plsc

N = 10000
E = 160000
F = 128
ED = 16
H = 6
C = 32
HC = H * C
G = 512
INV_SQRT_C = 1.0 / math.sqrt(C)

NC = 2
NS = 16
NW = NC * NS
ROW = 208
RB = 1000
NBLK = N // RB
EB = 2000
ZR = 624
HH = H // 2
HW = HH * C
ROW2 = 112


def _proj_body(x_ref, w_ref, b_ref, q_ref, kv_ref, skip_ref):
    full = jnp.dot(x_ref[...], w_ref[...], preferred_element_type=jnp.float32)
    full = full + b_ref[...]
    q_ref[0] = full[:, :HW]
    q_ref[1] = full[:, HW:HC]
    kv_ref[0] = jnp.concatenate(
        [full[:, HC:HC + HW], full[:, 2 * HC:2 * HC + HW]], axis=1)
    kv_ref[1] = jnp.concatenate(
        [full[:, HC + HW:2 * HC], full[:, 2 * HC + HW:3 * HC]], axis=1)
    skip_ref[...] = full[:, 3 * HC:]


def _proj(x, w, b):
    din = x.shape[1]
    return pl.pallas_call(
        _proj_body,
        grid=(NBLK,),
        in_specs=[
            pl.BlockSpec((RB, din), lambda i: (i, 0)),
            pl.BlockSpec((din, 4 * HC), lambda i: (0, 0)),
            pl.BlockSpec((1, 4 * HC), lambda i: (0, 0)),
        ],
        out_specs=[
            pl.BlockSpec((NC, RB, HW), lambda i: (0, i, 0)),
            pl.BlockSpec((NC, RB, 2 * HW), lambda i: (0, i, 0)),
            pl.BlockSpec((RB, HC), lambda i: (i, 0)),
        ],
        out_shape=[
            jax.ShapeDtypeStruct((NC, N, HW), jnp.float32),
            jax.ShapeDtypeStruct((NC, N, 2 * HW), jnp.float32),
            jax.ShapeDtypeStruct((N, HC), jnp.float32),
        ],
    )(x, w, b)


def _emat_body(ea_ref, we_ref, e1_ref, e2_ref, e3_ref):
    full = jnp.dot(ea_ref[...], we_ref[...], preferred_element_type=jnp.float32)
    for l, ref in enumerate([e1_ref, e2_ref, e3_ref]):
        ref[0] = full[:, l * HC:l * HC + HW]
        ref[1] = full[:, l * HC + HW:(l + 1) * HC]


def _emat(edge_attr, we3):
    return pl.pallas_call(
        _emat_body,
        grid=(E // EB,),
        in_specs=[
            pl.BlockSpec((EB, ED), lambda i: (i, 0)),
            pl.BlockSpec((ED, 3 * HC), lambda i: (0, 0)),
        ],
        out_specs=[pl.BlockSpec((NC, EB, HW), lambda i: (0, i, 0))] * 3,
        out_shape=[jax.ShapeDtypeStruct((NC, E, HW), jnp.float32)] * 3,
    )(edge_attr, we3)


def _postA_body(acc_ref, skip_ref, y_ref, st_ref):
    pieces = []
    for h in range(H):
        part = acc_ref[h // HH]
        hh = h % HH
        den = part[:, HW + hh:HW + hh + 1] + 1e-16
        pieces.append(part[:, hh * C:(hh + 1) * C] / den)
    y = jnp.concatenate(pieces, axis=1) + skip_ref[...]
    y_ref[...] = y
    st_ref[0, 0, :HC] = jnp.sum(y, axis=0)
    st_ref[0, 0, HC:] = jnp.sum(y * y, axis=0)


def _postA(accp, skip):
    return pl.pallas_call(
        _postA_body,
        grid=(NBLK,),
        in_specs=[
            pl.BlockSpec((NC, RB, ROW2), lambda i: (0, i, 0)),
            pl.BlockSpec((RB, HC), lambda i: (i, 0)),
        ],
        out_specs=[
            pl.BlockSpec((RB, HC), lambda i: (i, 0)),
            pl.BlockSpec((1, 1, 2 * HC), lambda i: (i, 0, 0)),
        ],
        out_shape=[
            jax.ShapeDtypeStruct((N, HC), jnp.float32),
            jax.ShapeDtypeStruct((NBLK, 1, 2 * HC), jnp.float32),
        ],
    )(accp, skip)


def _bn_from_stats(st):
    s = jnp.sum(st[:, 0, :HC], axis=0)
    sq = jnp.sum(st[:, 0, HC:], axis=0)
    m = (s / N).reshape(1, HC)
    var = (sq / N).reshape(1, HC) - m * m
    inv = lax.rsqrt(var + 1e-5)
    return m, inv


def _bnproj_body(y_ref, st_ref, res_ref, g_ref, bb_ref, w_ref, b_ref,
                 h_ref, q_ref, kv_ref, skip_ref):
    m, inv = _bn_from_stats(st_ref[...])
    hb = (y_ref[...] - m) * inv * g_ref[...] + bb_ref[...]
    hb = res_ref[...] + jnp.maximum(hb, 0.0)
    h_ref[...] = hb
    full = jnp.dot(hb, w_ref[...], preferred_element_type=jnp.float32)
    full = full + b_ref[...]
    q_ref[0] = full[:, :HW]
    q_ref[1] = full[:, HW:HC]
    kv_ref[0] = jnp.concatenate(
        [full[:, HC:HC + HW], full[:, 2 * HC:2 * HC + HW]], axis=1)
    kv_ref[1] = jnp.concatenate(
        [full[:, HC + HW:2 * HC], full[:, 2 * HC + HW:3 * HC]], axis=1)
    skip_ref[...] = full[:, 3 * HC:]


def _bnproj(y, st, resid, g, b, w, bias):
    return pl.pallas_call(
        _bnproj_body,
        grid=(NBLK,),
        in_specs=[
            pl.BlockSpec((RB, HC), lambda i: (i, 0)),
            pl.BlockSpec((NBLK, 1, 2 * HC), lambda i: (0, 0, 0)),
            pl.BlockSpec((RB, HC), lambda i: (i, 0)),
            pl.BlockSpec((1, HC), lambda i: (0, 0)),
            pl.BlockSpec((1, HC), lambda i: (0, 0)),
            pl.BlockSpec((HC, 4 * HC), lambda i: (0, 0)),
            pl.BlockSpec((1, 4 * HC), lambda i: (0, 0)),
        ],
        out_specs=[
            pl.BlockSpec((RB, HC), lambda i: (i, 0)),
            pl.BlockSpec((NC, RB, HW), lambda i: (0, i, 0)),
            pl.BlockSpec((NC, RB, 2 * HW), lambda i: (0, i, 0)),
            pl.BlockSpec((RB, HC), lambda i: (i, 0)),
        ],
        out_shape=[
            jax.ShapeDtypeStruct((N, HC), jnp.float32),
            jax.ShapeDtypeStruct((NC, N, HW), jnp.float32),
            jax.ShapeDtypeStruct((NC, N, 2 * HW), jnp.float32),
            jax.ShapeDtypeStruct((N, HC), jnp.float32),
        ],
    )(y, st, resid, g, b, w, bias)


def _gate_body(y_ref, st_ref, res_ref, g_ref, bb_ref, gw1_ref, gb1_ref,
               gw2_ref, gb2_ref, out_ref):
    m, inv = _bn_from_stats(st_ref[...])
    hb = (y_ref[...] - m) * inv * g_ref[...] + bb_ref[...]
    hb = res_ref[...] + jnp.maximum(hb, 0.0)
    t = jnp.dot(hb, gw1_ref[...], preferred_element_type=jnp.float32)
    t = jnp.maximum(t + gb1_ref[...], 0.0)
    t16 = t.astype(jnp.bfloat16).astype(jnp.float32)
    w16 = gw2_ref[...].astype(jnp.bfloat16).astype(jnp.float32)
    gate = jnp.sum(t16 * w16, axis=1, keepdims=True) + gb2_ref[0, 0]
    w = jnp.exp(gate)
    out_ref[...] = jnp.concatenate(
        [w * hb, w, jnp.zeros((RB, ROW - HC - 1), jnp.float32)], axis=1)


def _gate(y, st, resid, g, b, gw1, gb1, gw2row, gb2):
    return pl.pallas_call(
        _gate_body,
        grid=(NBLK,),
        in_specs=[
            pl.BlockSpec((RB, HC), lambda i: (i, 0)),
            pl.BlockSpec((NBLK, 1, 2 * HC), lambda i: (0, 0, 0)),
            pl.BlockSpec((RB, HC), lambda i: (i, 0)),
            pl.BlockSpec((1, HC), lambda i: (0, 0)),
            pl.BlockSpec((1, HC), lambda i: (0, 0)),
            pl.BlockSpec((HC, C), lambda i: (0, 0)),
            pl.BlockSpec((1, C), lambda i: (0, 0)),
            pl.BlockSpec((1, C), lambda i: (0, 0)),
            pl.BlockSpec((1, 1), lambda i: (0, 0)),
        ],
        out_specs=pl.BlockSpec((RB, ROW), lambda i: (i, 0)),
        out_shape=jax.ShapeDtypeStruct((N, ROW), jnp.float32),
    )(y, st, resid, g, b, gw1, gb1, gw2row, gb2)


def _pool_body(wh_ref, b_ref, acc_ref):
    bb = b_ref[0]
    iota = lax.broadcasted_iota(jnp.int32, (G, RB), 0)
    oh = (bb == iota).astype(jnp.float32)
    contrib = jnp.dot(oh, wh_ref[...], preferred_element_type=jnp.float32,
                      precision=lax.Precision.HIGHEST)

    @pl.when(pl.program_id(0) == 0)
    def _():
        acc_ref[...] = contrib

    @pl.when(pl.program_id(0) != 0)
    def _():
        acc_ref[...] = acc_ref[...] + contrib


def _pool(wh, batch3d):
    return pl.pallas_call(
        _pool_body,
        grid=(NBLK,),
        in_specs=[
            pl.BlockSpec((RB, ROW), lambda i: (i, 0)),
            pl.BlockSpec((1, 1, RB), lambda i: (i, 0, 0)),
        ],
        out_specs=pl.BlockSpec((G, ROW), lambda i: (0, 0)),
        out_shape=jax.ShapeDtypeStruct((G, ROW), jnp.float32),
    )(wh, batch3d)


def _final_body(acc_ref, fw1_ref, fb1_ref, fg1_ref, fbe1_ref,
                fw2_ref, fb2_ref, fg2_ref, fbe2_ref, fw3_ref, fb3_ref,
                out_ref):
    a = acc_ref[...]
    den = a[:, HC:HC + 1] + 1e-16
    pooled = a[:, :HC] / den

    def bn(t, gg, bb):
        mm = jnp.mean(t, axis=0, keepdims=True)
        ct = t - mm
        vv = jnp.mean(ct * ct, axis=0, keepdims=True)
        return ct * lax.rsqrt(vv + 1e-5) * gg + bb

    z = jnp.dot(pooled, fw1_ref[...], preferred_element_type=jnp.float32)
    z = jnp.maximum(bn(z + fb1_ref[...], fg1_ref[...], fbe1_ref[...]), 0.0)
    z = jnp.dot(z, fw2_ref[...], preferred_element_type=jnp.float32)
    z = jnp.maximum(bn(z + fb2_ref[...], fg2_ref[...], fbe2_ref[...]), 0.0)
    z16 = z.astype(jnp.bfloat16).astype(jnp.float32)
    f16 = fw3_ref[...].astype(jnp.bfloat16).astype(jnp.float32)
    out = jnp.sum(z16 * f16, axis=1, keepdims=True) + fb3_ref[0, 0]
    out_ref[...] = out


def _final(acc, fw1, fb1, fg1, fbe1, fw2, fb2, fg2, fbe2, fw3row, fb3):
    return pl.pallas_call(
        _final_body,
        grid=(1,),
        in_specs=[
            pl.BlockSpec((G, ROW), lambda i: (0, 0)),
            pl.BlockSpec((HC, 2 * C), lambda i: (0, 0)),
            pl.BlockSpec((1, 2 * C), lambda i: (0, 0)),
            pl.BlockSpec((1, 2 * C), lambda i: (0, 0)),
            pl.BlockSpec((1, 2 * C), lambda i: (0, 0)),
            pl.BlockSpec((2 * C, C), lambda i: (0, 0)),
            pl.BlockSpec((1, C), lambda i: (0, 0)),
            pl.BlockSpec((1, C), lambda i: (0, 0)),
            pl.BlockSpec((1, C), lambda i: (0, 0)),
            pl.BlockSpec((1, C), lambda i: (0, 0)),
            pl.BlockSpec((1, 1), lambda i: (0, 0)),
        ],
        out_specs=pl.BlockSpec((G, 1), lambda i: (0, 0)),
        out_shape=jax.ShapeDtypeStruct((G, 1), jnp.float32),
    )(acc, fw1, fb1, fg1, fbe1, fw2, fb2, fg2, fbe2, fw3row, fb3)


EPT = E // NS
BPT = EPT // 16
NSLOT = 3
NSS = 2


def _edge_body(qtab, kvtab, etab, s2h, d2h, accout,
               s2a, d2a, qd, kv, er, orr, db, zbuf, acc,
               semg, sems):
    cid = lax.axis_index("c")
    sid = lax.axis_index("s")

    zero16 = jnp.zeros((16,), jnp.float32)

    def zrow(i, _):
        for j in range(ROW2 // 16):
            zbuf[i, pl.ds(j * 16, 16)] = zero16
        return 0
    lax.fori_loop(0, 16, zrow, 0)
    for r in range(ZR // 16):
        pltpu.sync_copy(zbuf, acc.at[pl.ds(sid * ZR + r * 16, 16)])

    @pl.when(sid == NS - 1)
    def _():
        pltpu.sync_copy(zbuf, acc.at[pl.ds(NS * ZR, 16)])

    pltpu.sync_copy(s2h.at[cid, sid], s2a)
    pltpu.sync_copy(d2h.at[cid, sid], d2a)

    plsc.subcore_barrier()

    iot = lax.iota(jnp.int32, 16)
    noff = cid * N
    ebase = cid * E + sid * EPT

    def fetch(j, b):
        pltpu.async_copy(qtab.at[d2a.at[j]], qd[b], semg[b])
        pltpu.async_copy(kvtab.at[s2a.at[j]], kv[b], semg[b])
        pltpu.async_copy(etab.at[pl.ds(ebase + j * 16, 16)], er[b], semg[b])

    def process(i, b):
        ss = b & 1
        pltpu.make_async_copy(qtab.at[pl.ds(0, 16)], qd[b], semg[b]).wait()
        pltpu.make_async_copy(kvtab.at[pl.ds(0, 16)], kv[b], semg[b]).wait()
        pltpu.make_async_copy(etab.at[pl.ds(0, 16)], er[b], semg[b]).wait()

        parts = [[zero16, zero16] for _ in range(HH)]
        for c in range(C):
            for h in range(HH):
                fv = jnp.full((16,), h * C + c, jnp.int32)
                qT = plsc.load_gather(qd[b], [iot, fv])
                kT = plsc.load_gather(kv[b], [iot, fv])
                eT = plsc.load_gather(er[b], [iot, fv])
                parts[h][c % 2] = parts[h][c % 2] + qT * (kT + eT)
        exvs = [jnp.exp((p[0] + p[1]) * INV_SQRT_C) for p in parts]

        @pl.when(i >= NSS)
        def _():
            pltpu.make_async_copy(orr[ss], acc.at[db[ss]], sems[ss]).wait()

        for l in range(16):
            e0, e1, e2 = exvs[0][l], exvs[1][l], exvs[2][l]
            exrow = jnp.where(iot == 0, e0,
                              jnp.where(iot == 1, e1,
                                        jnp.where(iot == 2, e2, 0.0)))
            orr[ss][l, pl.ds(HW, 16)] = exrow
            for j in range(HW // 16):
                exs = (e0, e0, e1, e1, e2, e2)[j]
                vch = kv[b][l, pl.ds(HW + j * 16, 16)]
                ech = er[b][l, pl.ds(j * 16, 16)]
                orr[ss][l, pl.ds(j * 16, 16)] = (vch + ech) * exs

        db[ss][...] = d2a[i] - noff
        pltpu.async_copy(orr[ss], acc.at[db[ss]], sems[ss], add=True)

    for b in range(NSLOT):
        fetch(b, b)

    def outer(io, _):
        for b in range(NSLOT):
            i = NSLOT * io + b

            @pl.when(i < BPT)
            def _():
                process(i, b)

            @pl.when(i + NSLOT < BPT)
            def _():
                fetch(i + NSLOT, b)
        return 0

    lax.fori_loop(0, (BPT + NSLOT - 1) // NSLOT, outer, 0)

    for ss in range(NSS):
        pltpu.make_async_copy(orr[ss], acc.at[db[ss]], sems[ss]).wait()

    plsc.subcore_barrier()

    r0 = sid * ZR
    pltpu.sync_copy(acc.at[pl.ds(r0, ZR)], accout.at[cid, pl.ds(r0, ZR)])

    @pl.when(sid == NS - 1)
    def _():
        pltpu.sync_copy(acc.at[pl.ds(NS * ZR, 16)],
                        accout.at[cid, pl.ds(NS * ZR, 16)])


@functools.cache
def _get_edge_kernel():
    return pl.kernel(
        _edge_body,
        out_type=jax.ShapeDtypeStruct((NC, N, ROW2), jnp.float32),
        mesh=plsc.VectorSubcoreMesh(core_axis_name="c", subcore_axis_name="s"),
        compiler_params=pltpu.CompilerParams(use_tc_tiling_on_sc=False,
                                             needs_layout_passes=False),
        scratch_types=[
            pltpu.VMEM((BPT, 16), jnp.int32),
            pltpu.VMEM((BPT, 16), jnp.int32),
            [pltpu.VMEM((16, HW), jnp.float32) for _ in range(NSLOT)],
            [pltpu.VMEM((16, 2 * HW), jnp.float32) for _ in range(NSLOT)],
            [pltpu.VMEM((16, HW), jnp.float32) for _ in range(NSLOT)],
            [pltpu.VMEM((16, ROW2), jnp.float32) for _ in range(NSS)],
            [pltpu.VMEM((16,), jnp.int32) for _ in range(NSS)],
            pltpu.VMEM((16, ROW2), jnp.float32),
            pltpu.VMEM_SHARED((N, ROW2), jnp.float32),
            [pltpu.SemaphoreType.DMA for _ in range(NSLOT)],
            [pltpu.SemaphoreType.DMA for _ in range(NSS)],
        ],
    )


def kernel(x, edge_index, edge_attr, batch, params):
    p = params
    src = edge_index[0]
    dst = edge_index[1]
    batch3d = batch.reshape(NBLK, 1, RB)
    zeros_n = jnp.zeros((N, HC), jnp.float32)

    def wcat(pre):
        w = jnp.concatenate(
            [p[pre + 'Wq'], p[pre + 'Wk'], p[pre + 'Wv'], p[pre + 'Wskip']],
            axis=1)
        b = jnp.concatenate(
            [p[pre + 'bq'], p[pre + 'bk'], p[pre + 'bv'], p[pre + 'bskip']]
        ).reshape(1, 4 * HC)
        return w, b

    we3 = jnp.concatenate([p['c1We'], p['c2We'], p['c3We']], axis=1)
    e1, e2, e3 = _emat(edge_attr, we3)

    w1, b1 = wcat('c1')
    w2, b2 = wcat('c2')
    w3, b3 = wcat('c3')

    srcr = src.reshape(NS, BPT, 16)
    dstr = dst.reshape(NS, BPT, 16)
    s2h = jnp.stack([srcr, srcr + N])
    d2h = jnp.stack([dstr, dstr + N])

    def edge(q, kv, e):
        return _get_edge_kernel()(q.reshape(NC * N, HW),
                                  kv.reshape(NC * N, 2 * HW),
                                  e.reshape(NC * E, HW), s2h, d2h)

    q, kv, skip = _proj(x, w1, b1)
    accp = edge(q, kv, e1)
    y1, st1 = _postA(accp, skip)

    h1, q, kv, skip = _bnproj(y1, st1, zeros_n, p['bn1g'].reshape(1, HC),
                              p['bn1b'].reshape(1, HC), w2, b2)
    accp = edge(q, kv, e2)
    y2, st2 = _postA(accp, skip)

    h2, q, kv, skip = _bnproj(y2, st2, h1, p['bn2g'].reshape(1, HC),
                              p['bn2b'].reshape(1, HC), w3, b3)
    accp = edge(q, kv, e3)
    y3, st3 = _postA(accp, skip)

    wh = _gate(y3, st3, h2, p['bn3g'].reshape(1, HC), p['bn3b'].reshape(1, HC),
               p['gW1'], p['gb1'].reshape(1, C), p['gW2'].reshape(1, C),
               p['gb2'].reshape(1, 1))
    acc = _pool(wh, batch3d)

    out = _final(acc, p['fW1'], p['fb1'].reshape(1, 2 * C),
                 p['fg1'].reshape(1, 2 * C), p['fbe1'].reshape(1, 2 * C),
                 p['fW2'], p['fb2'].reshape(1, C), p['fg2'].reshape(1, C),
                 p['fbe2'].reshape(1, C), p['fW3'].reshape(1, C),
                 p['fb3'].reshape(1, 1))
    return out.reshape(G)


# speedup vs baseline: 1.2418x; 1.2418x over previous
import functools
import math

import jax
import jax.numpy as jnp
from jax import lax
from jax.experimental import pallas as pl
from jax.experimental.pallas import tpu as pltpu
from jax.experimental.pallas import tpu_sc as plsc

N = 10000
E = 160000
F = 128
ED = 16
H = 6
C = 32
HC = H * C
G = 512
INV_SQRT_C = 1.0 / math.sqrt(C)

NC = 2
NS = 16
NW = NC * NS
ROW = 208
RB = 1000
NBLK = N // RB
EB = 2000
ZR = 624
HH = H // 2
HW = HH * C
ROW2 = 112


def _proj_body(x_ref, w_ref, b_ref, q_ref, kv_ref, skip_ref):
    full = jnp.dot(x_ref[...], w_ref[...], preferred_element_type=jnp.float32)
    full = full + b_ref[...]
    q_ref[0] = full[:, :HW]
    q_ref[1] = full[:, HW:HC]
    kv_ref[0] = jnp.concatenate(
        [full[:, HC:HC + HW], full[:, 2 * HC:2 * HC + HW]], axis=1)
    kv_ref[1] = jnp.concatenate(
        [full[:, HC + HW:2 * HC], full[:, 2 * HC + HW:3 * HC]], axis=1)
    skip_ref[...] = full[:, 3 * HC:]


def _proj(x, w, b):
    din = x.shape[1]
    return pl.pallas_call(
        _proj_body,
        grid=(NBLK,),
        in_specs=[
            pl.BlockSpec((RB, din), lambda i: (i, 0)),
            pl.BlockSpec((din, 4 * HC), lambda i: (0, 0)),
            pl.BlockSpec((1, 4 * HC), lambda i: (0, 0)),
        ],
        out_specs=[
            pl.BlockSpec((NC, RB, HW), lambda i: (0, i, 0)),
            pl.BlockSpec((NC, RB, 2 * HW), lambda i: (0, i, 0)),
            pl.BlockSpec((RB, HC), lambda i: (i, 0)),
        ],
        out_shape=[
            jax.ShapeDtypeStruct((NC, N, HW), jnp.float32),
            jax.ShapeDtypeStruct((NC, N, 2 * HW), jnp.float32),
            jax.ShapeDtypeStruct((N, HC), jnp.float32),
        ],
    )(x, w, b)


def _emat_body(ea_ref, we_ref, e1_ref, e2_ref, e3_ref):
    full = jnp.dot(ea_ref[...], we_ref[...], preferred_element_type=jnp.float32)
    for l, ref in enumerate([e1_ref, e2_ref, e3_ref]):
        ref[0] = full[:, l * HC:l * HC + HW]
        ref[1] = full[:, l * HC + HW:(l + 1) * HC]


def _emat(edge_attr, we3):
    return pl.pallas_call(
        _emat_body,
        grid=(E // EB,),
        in_specs=[
            pl.BlockSpec((EB, ED), lambda i: (i, 0)),
            pl.BlockSpec((ED, 3 * HC), lambda i: (0, 0)),
        ],
        out_specs=[pl.BlockSpec((NC, EB, HW), lambda i: (0, i, 0))] * 3,
        out_shape=[jax.ShapeDtypeStruct((NC, E, HW), jnp.float32)] * 3,
    )(edge_attr, we3)


def _postA_body(acc_ref, skip_ref, y_ref, st_ref):
    pieces = []
    for h in range(H):
        part = acc_ref[h // HH]
        hh = h % HH
        den = part[:, HW + hh:HW + hh + 1] + 1e-16
        pieces.append(part[:, hh * C:(hh + 1) * C] / den)
    y = jnp.concatenate(pieces, axis=1) + skip_ref[...]
    y_ref[...] = y
    st_ref[0, 0, :HC] = jnp.sum(y, axis=0)
    st_ref[0, 0, HC:] = jnp.sum(y * y, axis=0)


def _postA(accp, skip):
    return pl.pallas_call(
        _postA_body,
        grid=(NBLK,),
        in_specs=[
            pl.BlockSpec((NC, RB, ROW2), lambda i: (0, i, 0)),
            pl.BlockSpec((RB, HC), lambda i: (i, 0)),
        ],
        out_specs=[
            pl.BlockSpec((RB, HC), lambda i: (i, 0)),
            pl.BlockSpec((1, 1, 2 * HC), lambda i: (i, 0, 0)),
        ],
        out_shape=[
            jax.ShapeDtypeStruct((N, HC), jnp.float32),
            jax.ShapeDtypeStruct((NBLK, 1, 2 * HC), jnp.float32),
        ],
    )(accp, skip)


def _bn_from_stats(st):
    s = jnp.sum(st[:, 0, :HC], axis=0)
    sq = jnp.sum(st[:, 0, HC:], axis=0)
    m = (s / N).reshape(1, HC)
    var = (sq / N).reshape(1, HC) - m * m
    inv = lax.rsqrt(var + 1e-5)
    return m, inv


def _bnproj_body(y_ref, st_ref, res_ref, g_ref, bb_ref, w_ref, b_ref,
                 h_ref, q_ref, kv_ref, skip_ref):
    m, inv = _bn_from_stats(st_ref[...])
    hb = (y_ref[...] - m) * inv * g_ref[...] + bb_ref[...]
    hb = res_ref[...] + jnp.maximum(hb, 0.0)
    h_ref[...] = hb
    full = jnp.dot(hb, w_ref[...], preferred_element_type=jnp.float32)
    full = full + b_ref[...]
    q_ref[0] = full[:, :HW]
    q_ref[1] = full[:, HW:HC]
    kv_ref[0] = jnp.concatenate(
        [full[:, HC:HC + HW], full[:, 2 * HC:2 * HC + HW]], axis=1)
    kv_ref[1] = jnp.concatenate(
        [full[:, HC + HW:2 * HC], full[:, 2 * HC + HW:3 * HC]], axis=1)
    skip_ref[...] = full[:, 3 * HC:]


def _bnproj(y, st, resid, g, b, w, bias):
    return pl.pallas_call(
        _bnproj_body,
        grid=(NBLK,),
        in_specs=[
            pl.BlockSpec((RB, HC), lambda i: (i, 0)),
            pl.BlockSpec((NBLK, 1, 2 * HC), lambda i: (0, 0, 0)),
            pl.BlockSpec((RB, HC), lambda i: (i, 0)),
            pl.BlockSpec((1, HC), lambda i: (0, 0)),
            pl.BlockSpec((1, HC), lambda i: (0, 0)),
            pl.BlockSpec((HC, 4 * HC), lambda i: (0, 0)),
            pl.BlockSpec((1, 4 * HC), lambda i: (0, 0)),
        ],
        out_specs=[
            pl.BlockSpec((RB, HC), lambda i: (i, 0)),
            pl.BlockSpec((NC, RB, HW), lambda i: (0, i, 0)),
            pl.BlockSpec((NC, RB, 2 * HW), lambda i: (0, i, 0)),
            pl.BlockSpec((RB, HC), lambda i: (i, 0)),
        ],
        out_shape=[
            jax.ShapeDtypeStruct((N, HC), jnp.float32),
            jax.ShapeDtypeStruct((NC, N, HW), jnp.float32),
            jax.ShapeDtypeStruct((NC, N, 2 * HW), jnp.float32),
            jax.ShapeDtypeStruct((N, HC), jnp.float32),
        ],
    )(y, st, resid, g, b, w, bias)


def _gate_body(y_ref, st_ref, res_ref, g_ref, bb_ref, gw1_ref, gb1_ref,
               gw2_ref, gb2_ref, out_ref):
    m, inv = _bn_from_stats(st_ref[...])
    hb = (y_ref[...] - m) * inv * g_ref[...] + bb_ref[...]
    hb = res_ref[...] + jnp.maximum(hb, 0.0)
    t = jnp.dot(hb, gw1_ref[...], preferred_element_type=jnp.float32)
    t = jnp.maximum(t + gb1_ref[...], 0.0)
    t16 = t.astype(jnp.bfloat16).astype(jnp.float32)
    w16 = gw2_ref[...].astype(jnp.bfloat16).astype(jnp.float32)
    gate = jnp.sum(t16 * w16, axis=1, keepdims=True) + gb2_ref[0, 0]
    w = jnp.exp(gate)
    out_ref[...] = jnp.concatenate(
        [w * hb, w, jnp.zeros((RB, ROW - HC - 1), jnp.float32)], axis=1)


def _gate(y, st, resid, g, b, gw1, gb1, gw2row, gb2):
    return pl.pallas_call(
        _gate_body,
        grid=(NBLK,),
        in_specs=[
            pl.BlockSpec((RB, HC), lambda i: (i, 0)),
            pl.BlockSpec((NBLK, 1, 2 * HC), lambda i: (0, 0, 0)),
            pl.BlockSpec((RB, HC), lambda i: (i, 0)),
            pl.BlockSpec((1, HC), lambda i: (0, 0)),
            pl.BlockSpec((1, HC), lambda i: (0, 0)),
            pl.BlockSpec((HC, C), lambda i: (0, 0)),
            pl.BlockSpec((1, C), lambda i: (0, 0)),
            pl.BlockSpec((1, C), lambda i: (0, 0)),
            pl.BlockSpec((1, 1), lambda i: (0, 0)),
        ],
        out_specs=pl.BlockSpec((RB, ROW), lambda i: (i, 0)),
        out_shape=jax.ShapeDtypeStruct((N, ROW), jnp.float32),
    )(y, st, resid, g, b, gw1, gb1, gw2row, gb2)


def _pool_body(wh_ref, b_ref, acc_ref):
    bb = b_ref[0]
    iota = lax.broadcasted_iota(jnp.int32, (G, RB), 0)
    oh = (bb == iota).astype(jnp.float32)
    contrib = jnp.dot(oh, wh_ref[...], preferred_element_type=jnp.float32,
                      precision=lax.Precision.HIGHEST)

    @pl.when(pl.program_id(0) == 0)
    def _():
        acc_ref[...] = contrib

    @pl.when(pl.program_id(0) != 0)
    def _():
        acc_ref[...] = acc_ref[...] + contrib


def _pool(wh, batch3d):
    return pl.pallas_call(
        _pool_body,
        grid=(NBLK,),
        in_specs=[
            pl.BlockSpec((RB, ROW), lambda i: (i, 0)),
            pl.BlockSpec((1, 1, RB), lambda i: (i, 0, 0)),
        ],
        out_specs=pl.BlockSpec((G, ROW), lambda i: (0, 0)),
        out_shape=jax.ShapeDtypeStruct((G, ROW), jnp.float32),
    )(wh, batch3d)


def _final_body(acc_ref, fw1_ref, fb1_ref, fg1_ref, fbe1_ref,
                fw2_ref, fb2_ref, fg2_ref, fbe2_ref, fw3_ref, fb3_ref,
                out_ref):
    a = acc_ref[...]
    den = a[:, HC:HC + 1] + 1e-16
    pooled = a[:, :HC] / den

    def bn(t, gg, bb):
        mm = jnp.mean(t, axis=0, keepdims=True)
        ct = t - mm
        vv = jnp.mean(ct * ct, axis=0, keepdims=True)
        return ct * lax.rsqrt(vv + 1e-5) * gg + bb

    z = jnp.dot(pooled, fw1_ref[...], preferred_element_type=jnp.float32)
    z = jnp.maximum(bn(z + fb1_ref[...], fg1_ref[...], fbe1_ref[...]), 0.0)
    z = jnp.dot(z, fw2_ref[...], preferred_element_type=jnp.float32)
    z = jnp.maximum(bn(z + fb2_ref[...], fg2_ref[...], fbe2_ref[...]), 0.0)
    z16 = z.astype(jnp.bfloat16).astype(jnp.float32)
    f16 = fw3_ref[...].astype(jnp.bfloat16).astype(jnp.float32)
    out = jnp.sum(z16 * f16, axis=1, keepdims=True) + fb3_ref[0, 0]
    out_ref[...] = out


def _final(acc, fw1, fb1, fg1, fbe1, fw2, fb2, fg2, fbe2, fw3row, fb3):
    return pl.pallas_call(
        _final_body,
        grid=(1,),
        in_specs=[
            pl.BlockSpec((G, ROW), lambda i: (0, 0)),
            pl.BlockSpec((HC, 2 * C), lambda i: (0, 0)),
            pl.BlockSpec((1, 2 * C), lambda i: (0, 0)),
            pl.BlockSpec((1, 2 * C), lambda i: (0, 0)),
            pl.BlockSpec((1, 2 * C), lambda i: (0, 0)),
            pl.BlockSpec((2 * C, C), lambda i: (0, 0)),
            pl.BlockSpec((1, C), lambda i: (0, 0)),
            pl.BlockSpec((1, C), lambda i: (0, 0)),
            pl.BlockSpec((1, C), lambda i: (0, 0)),
            pl.BlockSpec((1, C), lambda i: (0, 0)),
            pl.BlockSpec((1, 1), lambda i: (0, 0)),
        ],
        out_specs=pl.BlockSpec((G, 1), lambda i: (0, 0)),
        out_shape=jax.ShapeDtypeStruct((G, 1), jnp.float32),
    )(acc, fw1, fb1, fg1, fbe1, fw2, fb2, fg2, fbe2, fw3row, fb3)


EPT = E // NS
BPT = EPT // 16
NSLOT = 2
NSS = 2


def _edge_body(qtab, kvtab, etab, s2h, d2h, accout,
               s2a, d2a, qd, kv, er, orr, db, zbuf, acc,
               semg, sems):
    cid = lax.axis_index("c")
    sid = lax.axis_index("s")

    zero16 = jnp.zeros((16,), jnp.float32)

    def zrow(i, _):
        for j in range(ROW2 // 16):
            zbuf[i, pl.ds(j * 16, 16)] = zero16
        return 0
    lax.fori_loop(0, 16, zrow, 0)
    for r in range(ZR // 16):
        pltpu.sync_copy(zbuf, acc.at[pl.ds(sid * ZR + r * 16, 16)])

    @pl.when(sid == NS - 1)
    def _():
        pltpu.sync_copy(zbuf, acc.at[pl.ds(NS * ZR, 16)])

    pltpu.sync_copy(s2h.at[cid, sid], s2a)
    pltpu.sync_copy(d2h.at[cid, sid], d2a)

    plsc.subcore_barrier()

    iot = lax.iota(jnp.int32, 16)
    noff = cid * N
    ebase = cid * E + sid * EPT

    def fetch(j, b):
        pltpu.async_copy(qtab.at[d2a.at[j]], qd[b], semg[b])
        pltpu.async_copy(kvtab.at[s2a.at[j]], kv[b], semg[b])
        pltpu.async_copy(etab.at[pl.ds(ebase + j * 16, 16)], er[b], semg[b])

    def process(i, b):
        ss = b & 1
        pltpu.make_async_copy(qtab.at[pl.ds(0, 16)], qd[b], semg[b]).wait()
        pltpu.make_async_copy(kvtab.at[pl.ds(0, 16)], kv[b], semg[b]).wait()
        pltpu.make_async_copy(etab.at[pl.ds(0, 16)], er[b], semg[b]).wait()

        exvs = []
        for h in range(HH):
            parts = [zero16] * 8
            for c in range(C):
                fv = jnp.full((16,), h * C + c, jnp.int32)
                qT = plsc.load_gather(qd[b], [iot, fv])
                kT = plsc.load_gather(kv[b], [iot, fv])
                eT = plsc.load_gather(er[b], [iot, fv])
                parts[c % 8] = parts[c % 8] + qT * (kT + eT)
            al = ((parts[0] + parts[1]) + (parts[2] + parts[3])) + (
                (parts[4] + parts[5]) + (parts[6] + parts[7]))
            exvs.append(jnp.exp(al * INV_SQRT_C))

        @pl.when(i >= NSS)
        def _():
            pltpu.make_async_copy(orr[ss], acc.at[db[ss]], sems[ss]).wait()

        for l in range(16):
            e0, e1, e2 = exvs[0][l], exvs[1][l], exvs[2][l]
            exrow = jnp.where(iot == 0, e0,
                              jnp.where(iot == 1, e1,
                                        jnp.where(iot == 2, e2, 0.0)))
            orr[ss][l, pl.ds(HW, 16)] = exrow
            for j in range(HW // 16):
                exs = (e0, e0, e1, e1, e2, e2)[j]
                vch = kv[b][l, pl.ds(HW + j * 16, 16)]
                ech = er[b][l, pl.ds(j * 16, 16)]
                orr[ss][l, pl.ds(j * 16, 16)] = (vch + ech) * exs

        db[ss][...] = d2a[i] - noff
        pltpu.async_copy(orr[ss], acc.at[db[ss]], sems[ss], add=True)

    for b in range(NSLOT):
        fetch(b, b)

    def outer(io, _):
        for b in range(NSLOT):
            i = NSLOT * io + b

            @pl.when(i < BPT)
            def _():
                process(i, b)

            @pl.when(i + NSLOT < BPT)
            def _():
                fetch(i + NSLOT, b)
        return 0

    lax.fori_loop(0, (BPT + NSLOT - 1) // NSLOT, outer, 0)

    for ss in range(NSS):
        pltpu.make_async_copy(orr[ss], acc.at[db[ss]], sems[ss]).wait()

    plsc.subcore_barrier()

    r0 = sid * ZR
    pltpu.sync_copy(acc.at[pl.ds(r0, ZR)], accout.at[cid, pl.ds(r0, ZR)])

    @pl.when(sid == NS - 1)
    def _():
        pltpu.sync_copy(acc.at[pl.ds(NS * ZR, 16)],
                        accout.at[cid, pl.ds(NS * ZR, 16)])


@functools.cache
def _get_edge_kernel():
    return pl.kernel(
        _edge_body,
        out_type=jax.ShapeDtypeStruct((NC, N, ROW2), jnp.float32),
        mesh=plsc.VectorSubcoreMesh(core_axis_name="c", subcore_axis_name="s"),
        compiler_params=pltpu.CompilerParams(use_tc_tiling_on_sc=False,
                                             needs_layout_passes=False),
        scratch_types=[
            pltpu.VMEM((BPT, 16), jnp.int32),
            pltpu.VMEM((BPT, 16), jnp.int32),
            [pltpu.VMEM((16, HW), jnp.float32) for _ in range(NSLOT)],
            [pltpu.VMEM((16, 2 * HW), jnp.float32) for _ in range(NSLOT)],
            [pltpu.VMEM((16, HW), jnp.float32) for _ in range(NSLOT)],
            [pltpu.VMEM((16, ROW2), jnp.float32) for _ in range(NSS)],
            [pltpu.VMEM((16,), jnp.int32) for _ in range(NSS)],
            pltpu.VMEM((16, ROW2), jnp.float32),
            pltpu.VMEM_SHARED((N, ROW2), jnp.float32),
            [pltpu.SemaphoreType.DMA for _ in range(NSLOT)],
            [pltpu.SemaphoreType.DMA for _ in range(NSS)],
        ],
    )


def kernel(x, edge_index, edge_attr, batch, params):
    p = params
    src = edge_index[0]
    dst = edge_index[1]
    batch3d = batch.reshape(NBLK, 1, RB)
    zeros_n = jnp.zeros((N, HC), jnp.float32)

    def wcat(pre):
        w = jnp.concatenate(
            [p[pre + 'Wq'], p[pre + 'Wk'], p[pre + 'Wv'], p[pre + 'Wskip']],
            axis=1)
        b = jnp.concatenate(
            [p[pre + 'bq'], p[pre + 'bk'], p[pre + 'bv'], p[pre + 'bskip']]
        ).reshape(1, 4 * HC)
        return w, b

    we3 = jnp.concatenate([p['c1We'], p['c2We'], p['c3We']], axis=1)
    e1, e2, e3 = _emat(edge_attr, we3)

    w1, b1 = wcat('c1')
    w2, b2 = wcat('c2')
    w3, b3 = wcat('c3')

    srcr = src.reshape(NS, BPT, 16)
    dstr = dst.reshape(NS, BPT, 16)
    s2h = jnp.stack([srcr, srcr + N])
    d2h = jnp.stack([dstr, dstr + N])

    def edge(q, kv, e):
        return _get_edge_kernel()(q.reshape(NC * N, HW),
                                  kv.reshape(NC * N, 2 * HW),
                                  e.reshape(NC * E, HW), s2h, d2h)

    q, kv, skip = _proj(x, w1, b1)
    accp = edge(q, kv, e1)
    y1, st1 = _postA(accp, skip)

    h1, q, kv, skip = _bnproj(y1, st1, zeros_n, p['bn1g'].reshape(1, HC),
                              p['bn1b'].reshape(1, HC), w2, b2)
    accp = edge(q, kv, e2)
    y2, st2 = _postA(accp, skip)

    h2, q, kv, skip = _bnproj(y2, st2, h1, p['bn2g'].reshape(1, HC),
                              p['bn2b'].reshape(1, HC), w3, b3)
    accp = edge(q, kv, e3)
    y3, st3 = _postA(accp, skip)

    wh = _gate(y3, st3, h2, p['bn3g'].reshape(1, HC), p['bn3b'].reshape(1, HC),
               p['gW1'], p['gb1'].reshape(1, C), p['gW2'].reshape(1, C),
               p['gb2'].reshape(1, 1))
    acc = _pool(wh, batch3d)

    out = _final(acc, p['fW1'], p['fb1'].reshape(1, 2 * C),
                 p['fg1'].reshape(1, 2 * C), p['fbe1'].reshape(1, 2 * C),
                 p['fW2'], p['fb2'].reshape(1, C), p['fg2'].reshape(1, C),
                 p['fbe2'].reshape(1, C), p['fW3'].reshape(1, C),
                 p['fb3'].reshape(1, 1))
    return out.reshape(G)


# speedup vs baseline: 3.3751x; 2.7180x over previous
import functools
import math

import jax
import jax.numpy as jnp
from jax import lax
from jax.experimental import pallas as pl
from jax.experimental.pallas import tpu as pltpu
from jax.experimental.pallas import tpu_sc as plsc

N = 10000
E = 160000
F = 128
ED = 16
H = 6
C = 32
HC = H * C
G = 512
INV_SQRT_C = 1.0 / math.sqrt(C)

NC = 2
NS = 16
NW = NC * NS
ROW = 208
RB = 1000
NBLK = N // RB
EB = 2000
ZR = 624
HH = H // 2
HW = HH * C
ROW2 = 112


def _proj_body(x_ref, w_ref, b_ref, q_ref, kv_ref, skip_ref):
    full = jnp.dot(x_ref[...], w_ref[...], preferred_element_type=jnp.float32)
    full = full + b_ref[...]
    q_ref[0] = full[:, :HW]
    q_ref[1] = full[:, HW:HC]
    kv_ref[0] = jnp.concatenate(
        [full[:, HC:HC + HW], full[:, 2 * HC:2 * HC + HW]], axis=1)
    kv_ref[1] = jnp.concatenate(
        [full[:, HC + HW:2 * HC], full[:, 2 * HC + HW:3 * HC]], axis=1)
    skip_ref[...] = full[:, 3 * HC:]


def _proj(x, w, b):
    din = x.shape[1]
    return pl.pallas_call(
        _proj_body,
        grid=(NBLK,),
        in_specs=[
            pl.BlockSpec((RB, din), lambda i: (i, 0)),
            pl.BlockSpec((din, 4 * HC), lambda i: (0, 0)),
            pl.BlockSpec((1, 4 * HC), lambda i: (0, 0)),
        ],
        out_specs=[
            pl.BlockSpec((NC, RB, HW), lambda i: (0, i, 0)),
            pl.BlockSpec((NC, RB, 2 * HW), lambda i: (0, i, 0)),
            pl.BlockSpec((RB, HC), lambda i: (i, 0)),
        ],
        out_shape=[
            jax.ShapeDtypeStruct((NC, N, HW), jnp.float32),
            jax.ShapeDtypeStruct((NC, N, 2 * HW), jnp.float32),
            jax.ShapeDtypeStruct((N, HC), jnp.float32),
        ],
    )(x, w, b)


def _emat_body(ea_ref, we_ref, e1_ref, e2_ref, e3_ref):
    full = jnp.dot(ea_ref[...], we_ref[...], preferred_element_type=jnp.float32)
    for l, ref in enumerate([e1_ref, e2_ref, e3_ref]):
        ref[0] = full[:, l * HC:l * HC + HW]
        ref[1] = full[:, l * HC + HW:(l + 1) * HC]


def _emat(edge_attr, we3):
    return pl.pallas_call(
        _emat_body,
        grid=(E // EB,),
        in_specs=[
            pl.BlockSpec((EB, ED), lambda i: (i, 0)),
            pl.BlockSpec((ED, 3 * HC), lambda i: (0, 0)),
        ],
        out_specs=[pl.BlockSpec((NC, EB, HW), lambda i: (0, i, 0))] * 3,
        out_shape=[jax.ShapeDtypeStruct((NC, E, HW), jnp.float32)] * 3,
    )(edge_attr, we3)


def _postA_body(acc_ref, skip_ref, y_ref, st_ref):
    pieces = []
    for h in range(H):
        part = acc_ref[h // HH]
        hh = h % HH
        den = part[:, HW + hh:HW + hh + 1] + 1e-16
        pieces.append(part[:, hh * C:(hh + 1) * C] / den)
    y = jnp.concatenate(pieces, axis=1) + skip_ref[...]
    y_ref[...] = y
    st_ref[0, 0, :HC] = jnp.sum(y, axis=0)
    st_ref[0, 0, HC:] = jnp.sum(y * y, axis=0)


def _postA(accp, skip):
    return pl.pallas_call(
        _postA_body,
        grid=(NBLK,),
        in_specs=[
            pl.BlockSpec((NC, RB, ROW2), lambda i: (0, i, 0)),
            pl.BlockSpec((RB, HC), lambda i: (i, 0)),
        ],
        out_specs=[
            pl.BlockSpec((RB, HC), lambda i: (i, 0)),
            pl.BlockSpec((1, 1, 2 * HC), lambda i: (i, 0, 0)),
        ],
        out_shape=[
            jax.ShapeDtypeStruct((N, HC), jnp.float32),
            jax.ShapeDtypeStruct((NBLK, 1, 2 * HC), jnp.float32),
        ],
    )(accp, skip)


def _bn_from_stats(st):
    s = jnp.sum(st[:, 0, :HC], axis=0)
    sq = jnp.sum(st[:, 0, HC:], axis=0)
    m = (s / N).reshape(1, HC)
    var = (sq / N).reshape(1, HC) - m * m
    inv = lax.rsqrt(var + 1e-5)
    return m, inv


def _bnproj_body(y_ref, st_ref, res_ref, g_ref, bb_ref, w_ref, b_ref,
                 h_ref, q_ref, kv_ref, skip_ref):
    m, inv = _bn_from_stats(st_ref[...])
    hb = (y_ref[...] - m) * inv * g_ref[...] + bb_ref[...]
    hb = res_ref[...] + jnp.maximum(hb, 0.0)
    h_ref[...] = hb
    full = jnp.dot(hb, w_ref[...], preferred_element_type=jnp.float32)
    full = full + b_ref[...]
    q_ref[0] = full[:, :HW]
    q_ref[1] = full[:, HW:HC]
    kv_ref[0] = jnp.concatenate(
        [full[:, HC:HC + HW], full[:, 2 * HC:2 * HC + HW]], axis=1)
    kv_ref[1] = jnp.concatenate(
        [full[:, HC + HW:2 * HC], full[:, 2 * HC + HW:3 * HC]], axis=1)
    skip_ref[...] = full[:, 3 * HC:]


def _bnproj(y, st, resid, g, b, w, bias):
    return pl.pallas_call(
        _bnproj_body,
        grid=(NBLK,),
        in_specs=[
            pl.BlockSpec((RB, HC), lambda i: (i, 0)),
            pl.BlockSpec((NBLK, 1, 2 * HC), lambda i: (0, 0, 0)),
            pl.BlockSpec((RB, HC), lambda i: (i, 0)),
            pl.BlockSpec((1, HC), lambda i: (0, 0)),
            pl.BlockSpec((1, HC), lambda i: (0, 0)),
            pl.BlockSpec((HC, 4 * HC), lambda i: (0, 0)),
            pl.BlockSpec((1, 4 * HC), lambda i: (0, 0)),
        ],
        out_specs=[
            pl.BlockSpec((RB, HC), lambda i: (i, 0)),
            pl.BlockSpec((NC, RB, HW), lambda i: (0, i, 0)),
            pl.BlockSpec((NC, RB, 2 * HW), lambda i: (0, i, 0)),
            pl.BlockSpec((RB, HC), lambda i: (i, 0)),
        ],
        out_shape=[
            jax.ShapeDtypeStruct((N, HC), jnp.float32),
            jax.ShapeDtypeStruct((NC, N, HW), jnp.float32),
            jax.ShapeDtypeStruct((NC, N, 2 * HW), jnp.float32),
            jax.ShapeDtypeStruct((N, HC), jnp.float32),
        ],
    )(y, st, resid, g, b, w, bias)


def _gate_body(y_ref, st_ref, res_ref, g_ref, bb_ref, gw1_ref, gb1_ref,
               gw2_ref, gb2_ref, out_ref):
    m, inv = _bn_from_stats(st_ref[...])
    hb = (y_ref[...] - m) * inv * g_ref[...] + bb_ref[...]
    hb = res_ref[...] + jnp.maximum(hb, 0.0)
    t = jnp.dot(hb, gw1_ref[...], preferred_element_type=jnp.float32)
    t = jnp.maximum(t + gb1_ref[...], 0.0)
    t16 = t.astype(jnp.bfloat16).astype(jnp.float32)
    w16 = gw2_ref[...].astype(jnp.bfloat16).astype(jnp.float32)
    gate = jnp.sum(t16 * w16, axis=1, keepdims=True) + gb2_ref[0, 0]
    w = jnp.exp(gate)
    out_ref[...] = jnp.concatenate(
        [w * hb, w, jnp.zeros((RB, ROW - HC - 1), jnp.float32)], axis=1)


def _gate(y, st, resid, g, b, gw1, gb1, gw2row, gb2):
    return pl.pallas_call(
        _gate_body,
        grid=(NBLK,),
        in_specs=[
            pl.BlockSpec((RB, HC), lambda i: (i, 0)),
            pl.BlockSpec((NBLK, 1, 2 * HC), lambda i: (0, 0, 0)),
            pl.BlockSpec((RB, HC), lambda i: (i, 0)),
            pl.BlockSpec((1, HC), lambda i: (0, 0)),
            pl.BlockSpec((1, HC), lambda i: (0, 0)),
            pl.BlockSpec((HC, C), lambda i: (0, 0)),
            pl.BlockSpec((1, C), lambda i: (0, 0)),
            pl.BlockSpec((1, C), lambda i: (0, 0)),
            pl.BlockSpec((1, 1), lambda i: (0, 0)),
        ],
        out_specs=pl.BlockSpec((RB, ROW), lambda i: (i, 0)),
        out_shape=jax.ShapeDtypeStruct((N, ROW), jnp.float32),
    )(y, st, resid, g, b, gw1, gb1, gw2row, gb2)


def _pool_body(wh_ref, b_ref, acc_ref):
    bb = b_ref[0]
    iota = lax.broadcasted_iota(jnp.int32, (G, RB), 0)
    oh = (bb == iota).astype(jnp.float32)
    contrib = jnp.dot(oh, wh_ref[...], preferred_element_type=jnp.float32,
                      precision=lax.Precision.HIGHEST)

    @pl.when(pl.program_id(0) == 0)
    def _():
        acc_ref[...] = contrib

    @pl.when(pl.program_id(0) != 0)
    def _():
        acc_ref[...] = acc_ref[...] + contrib


def _pool(wh, batch3d):
    return pl.pallas_call(
        _pool_body,
        grid=(NBLK,),
        in_specs=[
            pl.BlockSpec((RB, ROW), lambda i: (i, 0)),
            pl.BlockSpec((1, 1, RB), lambda i: (i, 0, 0)),
        ],
        out_specs=pl.BlockSpec((G, ROW), lambda i: (0, 0)),
        out_shape=jax.ShapeDtypeStruct((G, ROW), jnp.float32),
    )(wh, batch3d)


def _final_body(acc_ref, fw1_ref, fb1_ref, fg1_ref, fbe1_ref,
                fw2_ref, fb2_ref, fg2_ref, fbe2_ref, fw3_ref, fb3_ref,
                out_ref):
    a = acc_ref[...]
    den = a[:, HC:HC + 1] + 1e-16
    pooled = a[:, :HC] / den

    def bn(t, gg, bb):
        mm = jnp.mean(t, axis=0, keepdims=True)
        ct = t - mm
        vv = jnp.mean(ct * ct, axis=0, keepdims=True)
        return ct * lax.rsqrt(vv + 1e-5) * gg + bb

    z = jnp.dot(pooled, fw1_ref[...], preferred_element_type=jnp.float32)
    z = jnp.maximum(bn(z + fb1_ref[...], fg1_ref[...], fbe1_ref[...]), 0.0)
    z = jnp.dot(z, fw2_ref[...], preferred_element_type=jnp.float32)
    z = jnp.maximum(bn(z + fb2_ref[...], fg2_ref[...], fbe2_ref[...]), 0.0)
    z16 = z.astype(jnp.bfloat16).astype(jnp.float32)
    f16 = fw3_ref[...].astype(jnp.bfloat16).astype(jnp.float32)
    out = jnp.sum(z16 * f16, axis=1, keepdims=True) + fb3_ref[0, 0]
    out_ref[...] = out


def _final(acc, fw1, fb1, fg1, fbe1, fw2, fb2, fg2, fbe2, fw3row, fb3):
    return pl.pallas_call(
        _final_body,
        grid=(1,),
        in_specs=[
            pl.BlockSpec((G, ROW), lambda i: (0, 0)),
            pl.BlockSpec((HC, 2 * C), lambda i: (0, 0)),
            pl.BlockSpec((1, 2 * C), lambda i: (0, 0)),
            pl.BlockSpec((1, 2 * C), lambda i: (0, 0)),
            pl.BlockSpec((1, 2 * C), lambda i: (0, 0)),
            pl.BlockSpec((2 * C, C), lambda i: (0, 0)),
            pl.BlockSpec((1, C), lambda i: (0, 0)),
            pl.BlockSpec((1, C), lambda i: (0, 0)),
            pl.BlockSpec((1, C), lambda i: (0, 0)),
            pl.BlockSpec((1, C), lambda i: (0, 0)),
            pl.BlockSpec((1, 1), lambda i: (0, 0)),
        ],
        out_specs=pl.BlockSpec((G, 1), lambda i: (0, 0)),
        out_shape=jax.ShapeDtypeStruct((G, 1), jnp.float32),
    )(acc, fw1, fb1, fg1, fbe1, fw2, fb2, fg2, fbe2, fw3row, fb3)


EPT = E // NS
BPT = EPT // 16
NSLOT = 2
NSS = 2


def _edge_body(qtab, kvtab, etab, s2h, d2h, accout,
               s2a, d2a, qd, kv, er, orr, db, zbuf, acc,
               semg, sems):
    cid = lax.axis_index("c")
    sid = lax.axis_index("s")

    zero16 = jnp.zeros((16,), jnp.float32)

    def zrow(i, _):
        for j in range(ROW2 // 16):
            zbuf[i, pl.ds(j * 16, 16)] = zero16
        return 0
    lax.fori_loop(0, 16, zrow, 0)
    for r in range(ZR // 16):
        pltpu.sync_copy(zbuf, acc.at[pl.ds(sid * ZR + r * 16, 16)])

    @pl.when(sid == NS - 1)
    def _():
        pltpu.sync_copy(zbuf, acc.at[pl.ds(NS * ZR, 16)])

    pltpu.sync_copy(s2h.at[cid, sid], s2a)
    pltpu.sync_copy(d2h.at[cid, sid], d2a)

    plsc.subcore_barrier()

    iot = lax.iota(jnp.int32, 16)
    noff = cid * N
    ebase = cid * E + sid * EPT

    def fetch(j, b):
        pltpu.async_copy(qtab.at[d2a.at[j]], qd[b], semg[b])
        pltpu.async_copy(kvtab.at[s2a.at[j]], kv[b], semg[b])
        pltpu.async_copy(etab.at[pl.ds(ebase + j * 16, 16)], er[b], semg[b])

    def process(i, b):
        ss = b & 1
        pltpu.make_async_copy(qtab.at[pl.ds(0, 16)], qd[b], semg[b]).wait()
        pltpu.make_async_copy(kvtab.at[pl.ds(0, 16)], kv[b], semg[b]).wait()
        pltpu.make_async_copy(etab.at[pl.ds(0, 16)], er[b], semg[b]).wait()

        @pl.when(i >= NSS)
        def _():
            pltpu.make_async_copy(orr[ss], acc.at[db[ss]], sems[ss]).wait()

        for l in range(16):
            ech = [er[b][l, pl.ds(j * 16, 16)] for j in range(HW // 16)]
            ahs = []
            for h in range(HH):
                j0, j1 = 2 * h, 2 * h + 1
                q0 = qd[b][l, pl.ds(j0 * 16, 16)]
                q1 = qd[b][l, pl.ds(j1 * 16, 16)]
                k0 = kv[b][l, pl.ds(j0 * 16, 16)]
                k1 = kv[b][l, pl.ds(j1 * 16, 16)]
                sh = q0 * (k0 + ech[j0]) + q1 * (k1 + ech[j1])
                ahs.append(jnp.sum(sh))
            arow = jnp.where(iot == 0, ahs[0],
                             jnp.where(iot == 1, ahs[1],
                                       jnp.where(iot == 2, ahs[2], 0.0)))
            exr = jnp.exp(arow * INV_SQRT_C)
            exrow = jnp.where(iot < HH, exr, 0.0)
            orr[ss][l, pl.ds(HW, 16)] = exrow
            e0, e1, e2 = exr[0], exr[1], exr[2]
            for j in range(HW // 16):
                vch = kv[b][l, pl.ds(HW + j * 16, 16)]
                exs = (e0, e0, e1, e1, e2, e2)[j]
                orr[ss][l, pl.ds(j * 16, 16)] = (vch + ech[j]) * exs

        db[ss][...] = d2a[i] - noff
        pltpu.async_copy(orr[ss], acc.at[db[ss]], sems[ss], add=True)

    for b in range(NSLOT):
        fetch(b, b)

    def outer(io, _):
        for b in range(NSLOT):
            i = NSLOT * io + b

            @pl.when(i < BPT)
            def _():
                process(i, b)

            @pl.when(i + NSLOT < BPT)
            def _():
                fetch(i + NSLOT, b)
        return 0

    lax.fori_loop(0, (BPT + NSLOT - 1) // NSLOT, outer, 0)

    for ss in range(NSS):
        pltpu.make_async_copy(orr[ss], acc.at[db[ss]], sems[ss]).wait()

    plsc.subcore_barrier()

    r0 = sid * ZR
    pltpu.sync_copy(acc.at[pl.ds(r0, ZR)], accout.at[cid, pl.ds(r0, ZR)])

    @pl.when(sid == NS - 1)
    def _():
        pltpu.sync_copy(acc.at[pl.ds(NS * ZR, 16)],
                        accout.at[cid, pl.ds(NS * ZR, 16)])


@functools.cache
def _get_edge_kernel():
    return pl.kernel(
        _edge_body,
        out_type=jax.ShapeDtypeStruct((NC, N, ROW2), jnp.float32),
        mesh=plsc.VectorSubcoreMesh(core_axis_name="c", subcore_axis_name="s"),
        compiler_params=pltpu.CompilerParams(use_tc_tiling_on_sc=False,
                                             needs_layout_passes=False),
        scratch_types=[
            pltpu.VMEM((BPT, 16), jnp.int32),
            pltpu.VMEM((BPT, 16), jnp.int32),
            [pltpu.VMEM((16, HW), jnp.float32) for _ in range(NSLOT)],
            [pltpu.VMEM((16, 2 * HW), jnp.float32) for _ in range(NSLOT)],
            [pltpu.VMEM((16, HW), jnp.float32) for _ in range(NSLOT)],
            [pltpu.VMEM((16, ROW2), jnp.float32) for _ in range(NSS)],
            [pltpu.VMEM((16,), jnp.int32) for _ in range(NSS)],
            pltpu.VMEM((16, ROW2), jnp.float32),
            pltpu.VMEM_SHARED((N, ROW2), jnp.float32),
            [pltpu.SemaphoreType.DMA for _ in range(NSLOT)],
            [pltpu.SemaphoreType.DMA for _ in range(NSS)],
        ],
    )


def kernel(x, edge_index, edge_attr, batch, params):
    p = params
    src = edge_index[0]
    dst = edge_index[1]
    batch3d = batch.reshape(NBLK, 1, RB)
    zeros_n = jnp.zeros((N, HC), jnp.float32)

    def wcat(pre):
        w = jnp.concatenate(
            [p[pre + 'Wq'], p[pre + 'Wk'], p[pre + 'Wv'], p[pre + 'Wskip']],
            axis=1)
        b = jnp.concatenate(
            [p[pre + 'bq'], p[pre + 'bk'], p[pre + 'bv'], p[pre + 'bskip']]
        ).reshape(1, 4 * HC)
        return w, b

    we3 = jnp.concatenate([p['c1We'], p['c2We'], p['c3We']], axis=1)
    e1, e2, e3 = _emat(edge_attr, we3)

    w1, b1 = wcat('c1')
    w2, b2 = wcat('c2')
    w3, b3 = wcat('c3')

    srcr = src.reshape(NS, BPT, 16)
    dstr = dst.reshape(NS, BPT, 16)
    s2h = jnp.stack([srcr, srcr + N])
    d2h = jnp.stack([dstr, dstr + N])

    def edge(q, kv, e):
        return _get_edge_kernel()(q.reshape(NC * N, HW),
                                  kv.reshape(NC * N, 2 * HW),
                                  e.reshape(NC * E, HW), s2h, d2h)

    q, kv, skip = _proj(x, w1, b1)
    accp = edge(q, kv, e1)
    y1, st1 = _postA(accp, skip)

    h1, q, kv, skip = _bnproj(y1, st1, zeros_n, p['bn1g'].reshape(1, HC),
                              p['bn1b'].reshape(1, HC), w2, b2)
    accp = edge(q, kv, e2)
    y2, st2 = _postA(accp, skip)

    h2, q, kv, skip = _bnproj(y2, st2, h1, p['bn2g'].reshape(1, HC),
                              p['bn2b'].reshape(1, HC), w3, b3)
    accp = edge(q, kv, e3)
    y3, st3 = _postA(accp, skip)

    wh = _gate(y3, st3, h2, p['bn3g'].reshape(1, HC), p['bn3b'].reshape(1, HC),
               p['gW1'], p['gb1'].reshape(1, C), p['gW2'].reshape(1, C),
               p['gb2'].reshape(1, 1))
    acc = _pool(wh, batch3d)

    out = _final(acc, p['fW1'], p['fb1'].reshape(1, 2 * C),
                 p['fg1'].reshape(1, 2 * C), p['fbe1'].reshape(1, 2 * C),
                 p['fW2'], p['fb2'].reshape(1, C), p['fg2'].reshape(1, C),
                 p['fbe2'].reshape(1, C), p['fW3'].reshape(1, C),
                 p['fb3'].reshape(1, 1))
    return out.reshape(G)
